# Initial kernel scaffold; baseline (speedup 1.0000x reference)
import jax
import jax.numpy as jnp
from jax.experimental import pallas as pl


def kernel(pred_logits, pred_boxes, obj, target_sizes):
    raise NotImplementedError("write your pallas kernel here")


# speedup vs baseline: 5.7404x; 5.7404x over previous
import jax
import jax.numpy as jnp
from jax.experimental import pallas as pl
from jax.experimental.pallas import tpu as pltpu

_N = 20000
_C = 91
_K = 100
_CHUNK = 160
_NCHUNK = _N // _CHUNK


def _pp_kernel(logits_hbm, boxes_hbm, ts_ref,
               scores_ref, labels_ref, boxes_ref,
               s_ref, bx_ref, cmax_ref, sem1, sem2):
    b = pl.program_id(0)

    cp1 = pltpu.make_async_copy(logits_hbm.at[b], s_ref, sem1)
    cp1.start()
    cp2 = pltpu.make_async_copy(boxes_hbm.at[b], bx_ref, sem2)
    cp2.start()
    cp1.wait()

    lane128 = jax.lax.broadcasted_iota(jnp.int32, (1, 128), 1)
    cmax_ref[0:1, :] = jnp.full((1, 128), -1.0, jnp.float32)

    def stage(i, _):
        blk = s_ref[pl.ds(i * _CHUNK, _CHUNK), :]
        sb = jax.nn.sigmoid(blk)
        s_ref[pl.ds(i * _CHUNK, _CHUNK), :] = sb
        cm = jnp.max(sb)
        cmax_ref[0:1, :] = jnp.where(lane128 == i, cm, cmax_ref[0:1, :])
        return 0

    jax.lax.fori_loop(0, _NCHUNK, stage, 0)
    cp2.wait()

    h_img = ts_ref[0, 0, 0]
    w_img = ts_ref[0, 0, 1]

    lane100 = jax.lax.broadcasted_iota(jnp.int32, (1, _K), 1)
    row_i = jax.lax.broadcasted_iota(jnp.int32, (_CHUNK, _C), 0)
    col_i = jax.lax.broadcasted_iota(jnp.int32, (_CHUNK, _C), 1)
    row8 = jax.lax.broadcasted_iota(jnp.int32, (8, _C), 0)
    col8 = jax.lax.broadcasted_iota(jnp.int32, (8, _C), 1)
    brow8 = jax.lax.broadcasted_iota(jnp.int32, (8, 128), 0)
    bcol8 = jax.lax.broadcasted_iota(jnp.int32, (8, 128), 1)

    def extract(k, _):
        cmv = cmax_ref[0:1, :]
        m = jnp.max(cmv)
        ch = jnp.min(jnp.where(cmv == m, lane128, jnp.int32(2**31 - 1)))
        base = ch * _CHUNK
        blk = s_ref[pl.ds(base, _CHUNK), :]
        lin = (base + row_i) * _C + col_i
        idx = jnp.min(jnp.where(blk == m, lin, jnp.int32(2**31 - 1)))
        n = idx // _C
        lab = idx - n * _C

        scores_ref[0, 0:1, :] = jnp.where(lane100 == k, m, scores_ref[0, 0:1, :])
        labels_ref[0, 0:1, :] = jnp.where(lane100 == k, lab, labels_ref[0, 0:1, :])

        n8 = (n // 8) * 8
        slab = s_ref[pl.ds(n8, 8), :]
        hit = (row8 == (n - n8)) & (col8 == lab)
        s_ref[pl.ds(n8, 8), :] = jnp.where(hit, -1.0, slab)

        cm2 = jnp.max(s_ref[pl.ds(base, _CHUNK), :])
        cmax_ref[0:1, :] = jnp.where(lane128 == ch, cm2, cmax_ref[0:1, :])

        r4 = n // 32
        l4 = (n - r4 * 32) * 4
        r8 = (r4 // 8) * 8
        bslab = bx_ref[pl.ds(r8, 8), :]
        rhit = brow8 == (r4 - r8)

        def comp(j):
            return jnp.sum(jnp.where(rhit & (bcol8 == l4 + j), bslab, 0.0))

        cx, cy, w, h = comp(0), comp(1), comp(2), comp(3)
        x0 = (cx - 0.5 * w) * w_img
        y0 = (cy - 0.5 * h) * h_img
        x1 = (cx + 0.5 * w) * w_img
        y1 = (cy + 0.5 * h) * h_img
        boxes_ref[0, 0, 0:1, :] = jnp.where(lane100 == k, x0, boxes_ref[0, 0, 0:1, :])
        boxes_ref[0, 0, 1:2, :] = jnp.where(lane100 == k, y0, boxes_ref[0, 0, 1:2, :])
        boxes_ref[0, 0, 2:3, :] = jnp.where(lane100 == k, x1, boxes_ref[0, 0, 2:3, :])
        boxes_ref[0, 0, 3:4, :] = jnp.where(lane100 == k, y1, boxes_ref[0, 0, 3:4, :])
        return 0

    jax.lax.fori_loop(0, _K, extract, 0)


def kernel(pred_logits, pred_boxes, obj, target_sizes):
    B, N, C = pred_logits.shape
    bx = pred_boxes.reshape(B, (N * 4) // 128, 128)
    ts = target_sizes.astype(jnp.float32).reshape(B, 1, 2)

    scores, labels, boxes = pl.pallas_call(
        _pp_kernel,
        grid=(B,),
        in_specs=[
            pl.BlockSpec(memory_space=pl.ANY),
            pl.BlockSpec(memory_space=pl.ANY),
            pl.BlockSpec((1, 1, 2), lambda b: (b, 0, 0), memory_space=pltpu.SMEM),
        ],
        out_specs=[
            pl.BlockSpec((1, 1, _K), lambda b: (b, 0, 0)),
            pl.BlockSpec((1, 1, _K), lambda b: (b, 0, 0)),
            pl.BlockSpec((1, 1, 4, _K), lambda b: (b, 0, 0, 0)),
        ],
        out_shape=[
            jax.ShapeDtypeStruct((B, 1, _K), jnp.float32),
            jax.ShapeDtypeStruct((B, 1, _K), jnp.int32),
            jax.ShapeDtypeStruct((B, 1, 4, _K), jnp.float32),
        ],
        scratch_shapes=[
            pltpu.VMEM((_N, _C), jnp.float32),
            pltpu.VMEM(((N * 4) // 128, 128), jnp.float32),
            pltpu.VMEM((1, 128), jnp.float32),
            pltpu.SemaphoreType.DMA,
            pltpu.SemaphoreType.DMA,
        ],
    )(pred_logits, bx, ts)

    scores = scores.reshape(B, _K)
    labels = labels.reshape(B, _K)
    boxes = boxes.reshape(B, 4, _K).transpose(0, 2, 1)
    return scores, labels, boxes


# speedup vs baseline: 6.0292x; 1.0503x over previous
import jax
import jax.numpy as jnp
from jax.experimental import pallas as pl
from jax.experimental.pallas import tpu as pltpu

_N = 20000
_C = 91
_K = 100
_CHUNK = 80
_STAGE = 160
_NSTAGE = _N // _STAGE
_INT_INF = 2**31 - 1


def _pp_kernel(logits_hbm, boxes_hbm, ts_ref,
               scores_ref, labels_ref, boxes_ref,
               s_ref, bx_ref, cmax_ref, sem1, sem2):
    b = pl.program_id(0)
    nb = pl.num_programs(0)
    p = jax.lax.rem(b, 2)

    @pl.when(b == 0)
    def _():
        pltpu.make_async_copy(logits_hbm.at[0], s_ref.at[0], sem1).start()
        pltpu.make_async_copy(boxes_hbm.at[0], bx_ref.at[0], sem2).start()

    pltpu.make_async_copy(logits_hbm.at[b], s_ref.at[p], sem1).wait()
    pltpu.make_async_copy(boxes_hbm.at[b], bx_ref.at[p], sem2).wait()

    @pl.when(b + 1 < nb)
    def _():
        pltpu.make_async_copy(logits_hbm.at[b + 1], s_ref.at[1 - p], sem1).start()
        pltpu.make_async_copy(boxes_hbm.at[b + 1], bx_ref.at[1 - p], sem2).start()

    sv = s_ref.at[p]
    bv = bx_ref.at[p]

    row2 = jax.lax.broadcasted_iota(jnp.int32, (2, 128), 0)
    lane2 = jax.lax.broadcasted_iota(jnp.int32, (2, 128), 1)
    cmax_ref[0:2, :] = jnp.full((2, 128), -1.0, jnp.float32)

    def stage(i, _):
        blk = sv[pl.ds(i * _STAGE, _STAGE), :]
        sb = jax.nn.sigmoid(blk)
        sv[pl.ds(i * _STAGE, _STAGE), :] = sb
        cma = jnp.max(sb[0:_CHUNK])
        cmb = jnp.max(sb[_CHUNK:_STAGE])
        hit = lane2 == i
        cm = jnp.where(row2 == 0, cma, cmb)
        cmax_ref[0:2, :] = jnp.where(hit, cm, cmax_ref[0:2, :])
        return 0

    jax.lax.fori_loop(0, _NSTAGE, stage, 0)

    h_img = ts_ref[0, 0, 0]
    w_img = ts_ref[0, 0, 1]

    lane100 = jax.lax.broadcasted_iota(jnp.int32, (1, _K), 1)
    row_i = jax.lax.broadcasted_iota(jnp.int32, (_CHUNK, _C), 0)
    col_i = jax.lax.broadcasted_iota(jnp.int32, (_CHUNK, _C), 1)
    row8 = jax.lax.broadcasted_iota(jnp.int32, (8, _C), 0)
    col8 = jax.lax.broadcasted_iota(jnp.int32, (8, _C), 1)
    brow8 = jax.lax.broadcasted_iota(jnp.int32, (8, 128), 0)
    bcol8 = jax.lax.broadcasted_iota(jnp.int32, (8, 128), 1)

    def extract(k, _):
        cmv = cmax_ref[0:2, :]
        m = jnp.max(cmv)
        cid = jnp.min(jnp.where(cmv == m, row2 + 2 * lane2, _INT_INF))
        base = cid * _CHUNK
        blk = sv[pl.ds(base, _CHUNK), :]
        lin = (base + row_i) * _C + col_i
        idx = jnp.min(jnp.where(blk == m, lin, _INT_INF))
        n = idx // _C
        lab = idx - n * _C

        scores_ref[0, 0:1, :] = jnp.where(lane100 == k, m, scores_ref[0, 0:1, :])
        labels_ref[0, 0:1, :] = jnp.where(lane100 == k, lab, labels_ref[0, 0:1, :])

        cm2 = jnp.max(jnp.where(lin == idx, -1.0, blk))
        cl = cid // 2
        cr = cid - cl * 2
        cmax_ref[0:2, :] = jnp.where((lane2 == cl) & (row2 == cr), cm2,
                                     cmax_ref[0:2, :])

        n8 = (n // 8) * 8
        slab = sv[pl.ds(n8, 8), :]
        sv[pl.ds(n8, 8), :] = jnp.where((row8 == n - n8) & (col8 == lab),
                                        -1.0, slab)

        r4 = n // 32
        l4 = (n - r4 * 32) * 4
        r8 = (r4 // 8) * 8
        bslab = bv[pl.ds(r8, 8), :]
        rhit = brow8 == (r4 - r8)

        def comp(j):
            return jnp.sum(jnp.where(rhit & (bcol8 == l4 + j), bslab, 0.0))

        cx, cy, w, h = comp(0), comp(1), comp(2), comp(3)
        x0 = (cx - 0.5 * w) * w_img
        y0 = (cy - 0.5 * h) * h_img
        x1 = (cx + 0.5 * w) * w_img
        y1 = (cy + 0.5 * h) * h_img
        boxes_ref[0, 0, 0:1, :] = jnp.where(lane100 == k, x0, boxes_ref[0, 0, 0:1, :])
        boxes_ref[0, 0, 1:2, :] = jnp.where(lane100 == k, y0, boxes_ref[0, 0, 1:2, :])
        boxes_ref[0, 0, 2:3, :] = jnp.where(lane100 == k, x1, boxes_ref[0, 0, 2:3, :])
        boxes_ref[0, 0, 3:4, :] = jnp.where(lane100 == k, y1, boxes_ref[0, 0, 3:4, :])
        return 0

    jax.lax.fori_loop(0, _K, extract, 0)


def kernel(pred_logits, pred_boxes, obj, target_sizes):
    B, N, C = pred_logits.shape
    bx = pred_boxes.reshape(B, (N * 4) // 128, 128)
    ts = target_sizes.astype(jnp.float32).reshape(B, 1, 2)

    scores, labels, boxes = pl.pallas_call(
        _pp_kernel,
        grid=(B,),
        in_specs=[
            pl.BlockSpec(memory_space=pl.ANY),
            pl.BlockSpec(memory_space=pl.ANY),
            pl.BlockSpec((1, 1, 2), lambda b: (b, 0, 0), memory_space=pltpu.SMEM),
        ],
        out_specs=[
            pl.BlockSpec((1, 1, _K), lambda b: (b, 0, 0)),
            pl.BlockSpec((1, 1, _K), lambda b: (b, 0, 0)),
            pl.BlockSpec((1, 1, 4, _K), lambda b: (b, 0, 0, 0)),
        ],
        out_shape=[
            jax.ShapeDtypeStruct((B, 1, _K), jnp.float32),
            jax.ShapeDtypeStruct((B, 1, _K), jnp.int32),
            jax.ShapeDtypeStruct((B, 1, 4, _K), jnp.float32),
        ],
        scratch_shapes=[
            pltpu.VMEM((2, _N, _C), jnp.float32),
            pltpu.VMEM((2, (N * 4) // 128, 128), jnp.float32),
            pltpu.VMEM((2, 128), jnp.float32),
            pltpu.SemaphoreType.DMA,
            pltpu.SemaphoreType.DMA,
        ],
    )(pred_logits, bx, ts)

    scores = scores.reshape(B, _K)
    labels = labels.reshape(B, _K)
    boxes = boxes.reshape(B, 4, _K).transpose(0, 2, 1)
    return scores, labels, boxes


# speedup vs baseline: 7.0729x; 1.1731x over previous
import jax
import jax.numpy as jnp
from jax.experimental import pallas as pl
from jax.experimental.pallas import tpu as pltpu

_N = 20000
_C = 91
_K = 100
_CHUNK = 80
_STAGE = 160
_NSTAGE = _N // _STAGE
_INT_INF = 2**31 - 1


def _pp_kernel(logits_hbm, boxes_hbm, ts_ref,
               scores_ref, labels_ref, boxes_ref,
               s_ref, bx_ref, sem1, sem2):
    b = pl.program_id(0)
    nb = pl.num_programs(0)
    p = jax.lax.rem(b, 2)

    @pl.when(b == 0)
    def _():
        pltpu.make_async_copy(logits_hbm.at[0], s_ref.at[0], sem1).start()
        pltpu.make_async_copy(boxes_hbm.at[0], bx_ref.at[0], sem2).start()

    pltpu.make_async_copy(logits_hbm.at[b], s_ref.at[p], sem1).wait()
    pltpu.make_async_copy(boxes_hbm.at[b], bx_ref.at[p], sem2).wait()

    @pl.when(b + 1 < nb)
    def _():
        pltpu.make_async_copy(logits_hbm.at[b + 1], s_ref.at[1 - p], sem1).start()
        pltpu.make_async_copy(boxes_hbm.at[b + 1], bx_ref.at[1 - p], sem2).start()

    sv = s_ref.at[p]
    bv = bx_ref.at[p]

    row2 = jax.lax.broadcasted_iota(jnp.int32, (2, 128), 0)
    lane2 = jax.lax.broadcasted_iota(jnp.int32, (2, 128), 1)

    def stage(i, cmax):
        blk = sv[pl.ds(i * _STAGE, _STAGE), :]
        sb = jax.nn.sigmoid(blk)
        sv[pl.ds(i * _STAGE, _STAGE), :] = sb
        cma = jnp.max(sb[0:_CHUNK])
        cmb = jnp.max(sb[_CHUNK:_STAGE])
        cm = jnp.where(row2 == 0, cma, cmb)
        return jnp.where(lane2 == i, cm, cmax)

    cmax0 = jax.lax.fori_loop(0, _NSTAGE, stage,
                              jnp.full((2, 128), -1.0, jnp.float32),
                              unroll=5)

    h_img = ts_ref[0, 0, 0]
    w_img = ts_ref[0, 0, 1]

    lane128 = jax.lax.broadcasted_iota(jnp.int32, (1, 128), 1)
    row_i = jax.lax.broadcasted_iota(jnp.int32, (_CHUNK, _C), 0)
    col_i = jax.lax.broadcasted_iota(jnp.int32, (_CHUNK, _C), 1)
    row8 = jax.lax.broadcasted_iota(jnp.int32, (8, _C), 0)
    col8 = jax.lax.broadcasted_iota(jnp.int32, (8, _C), 1)
    brow8 = jax.lax.broadcasted_iota(jnp.int32, (8, 128), 0)
    bcol8 = jax.lax.broadcasted_iota(jnp.int32, (8, 128), 1)

    zf = jnp.zeros((1, 128), jnp.float32)
    zi = jnp.zeros((1, 128), jnp.int32)

    def extract(k, carry):
        cmax, sc_v, lb_v, b0, b1, b2, b3 = carry
        m = jnp.max(cmax)
        cid = jnp.min(jnp.where(cmax == m, row2 + 2 * lane2, _INT_INF))
        base = cid * _CHUNK
        blk = sv[pl.ds(base, _CHUNK), :]
        lin = (base + row_i) * _C + col_i
        idx = jnp.min(jnp.where(blk == m, lin, _INT_INF))
        n = idx // _C
        lab = idx - n * _C

        khit = lane128 == k
        sc_v = jnp.where(khit, m, sc_v)
        lb_v = jnp.where(khit, lab, lb_v)

        cm2 = jnp.max(jnp.where(lin == idx, -1.0, blk))
        cl = cid // 2
        cr = cid - cl * 2
        cmax = jnp.where((lane2 == cl) & (row2 == cr), cm2, cmax)

        n8 = (n // 8) * 8
        slab = sv[pl.ds(n8, 8), :]
        sv[pl.ds(n8, 8), :] = jnp.where((row8 == n - n8) & (col8 == lab),
                                        -1.0, slab)

        r4 = n // 32
        l4 = (n - r4 * 32) * 4
        r8 = (r4 // 8) * 8
        bslab = bv[pl.ds(r8, 8), :]
        rhit = brow8 == (r4 - r8)

        def comp(j):
            return jnp.sum(jnp.where(rhit & (bcol8 == l4 + j), bslab, 0.0))

        cx, cy, w, h = comp(0), comp(1), comp(2), comp(3)
        b0 = jnp.where(khit, (cx - 0.5 * w) * w_img, b0)
        b1 = jnp.where(khit, (cy - 0.5 * h) * h_img, b1)
        b2 = jnp.where(khit, (cx + 0.5 * w) * w_img, b2)
        b3 = jnp.where(khit, (cy + 0.5 * h) * h_img, b3)
        return cmax, sc_v, lb_v, b0, b1, b2, b3

    _, sc_v, lb_v, b0, b1, b2, b3 = jax.lax.fori_loop(
        0, _K, extract, (cmax0, zf, zi, zf, zf, zf, zf), unroll=2)

    scores_ref[0, 0:1, :] = sc_v[:, :_K]
    labels_ref[0, 0:1, :] = lb_v[:, :_K]
    boxes_ref[0, 0, 0:1, :] = b0[:, :_K]
    boxes_ref[0, 0, 1:2, :] = b1[:, :_K]
    boxes_ref[0, 0, 2:3, :] = b2[:, :_K]
    boxes_ref[0, 0, 3:4, :] = b3[:, :_K]


def kernel(pred_logits, pred_boxes, obj, target_sizes):
    B, N, C = pred_logits.shape
    bx = pred_boxes.reshape(B, (N * 4) // 128, 128)
    ts = target_sizes.astype(jnp.float32).reshape(B, 1, 2)

    scores, labels, boxes = pl.pallas_call(
        _pp_kernel,
        grid=(B,),
        in_specs=[
            pl.BlockSpec(memory_space=pl.ANY),
            pl.BlockSpec(memory_space=pl.ANY),
            pl.BlockSpec((1, 1, 2), lambda b: (b, 0, 0), memory_space=pltpu.SMEM),
        ],
        out_specs=[
            pl.BlockSpec((1, 1, _K), lambda b: (b, 0, 0)),
            pl.BlockSpec((1, 1, _K), lambda b: (b, 0, 0)),
            pl.BlockSpec((1, 1, 4, _K), lambda b: (b, 0, 0, 0)),
        ],
        out_shape=[
            jax.ShapeDtypeStruct((B, 1, _K), jnp.float32),
            jax.ShapeDtypeStruct((B, 1, _K), jnp.int32),
            jax.ShapeDtypeStruct((B, 1, 4, _K), jnp.float32),
        ],
        scratch_shapes=[
            pltpu.VMEM((2, _N, _C), jnp.float32),
            pltpu.VMEM((2, (N * 4) // 128, 128), jnp.float32),
            pltpu.SemaphoreType.DMA,
            pltpu.SemaphoreType.DMA,
        ],
    )(pred_logits, bx, ts)

    scores = scores.reshape(B, _K)
    labels = labels.reshape(B, _K)
    boxes = boxes.reshape(B, 4, _K).transpose(0, 2, 1)
    return scores, labels, boxes
